# baseline (speedup 1.0000x reference)
import functools

import jax
import jax.numpy as jnp
from jax import lax
from jax.experimental import pallas as pl
from jax.experimental.pallas import tpu as pltpu
from jax.experimental.pallas import tpu_sc as plsc

NUM_CORES = 2
NUM_SUBCORES = 16
NUM_WORKERS = NUM_CORES * NUM_SUBCORES
LANES = 16


def _body(b_per_w, K, u_hbm, b_hbm, w_hbm, W_hbm, V_hbm, outb_hbm, outw_hbm,
          uidx_v, bidx_v, widx_v, wu_v, vb_v, vw_v, sem_u, sem_b, sem_w):
    wid = lax.axis_index("s") * NUM_CORES + lax.axis_index("c")
    base = wid * b_per_w
    pltpu.sync_copy(u_hbm.at[pl.ds(base, b_per_w)], uidx_v)
    pltpu.sync_copy(b_hbm.at[pl.ds(base, b_per_w)], bidx_v)
    pltpu.sync_copy(w_hbm.at[pl.ds(base, b_per_w)], widx_v)
    cp_u = pltpu.async_copy(W_hbm.at[uidx_v], wu_v, sem_u)
    cp_b = pltpu.async_copy(V_hbm.at[bidx_v], vb_v, sem_b)
    cp_w = pltpu.async_copy(V_hbm.at[widx_v], vw_v, sem_w)
    cp_u.wait()
    cp_b.wait()
    cp_w.wait()

    def mul_row(r, carry):
        for j in range(K // LANES):
            sl = pl.ds(j * LANES, LANES)
            wu = wu_v[r, sl]
            vb_v[r, sl] = wu * vb_v[r, sl]
            vw_v[r, sl] = wu * vw_v[r, sl]
        return carry

    lax.fori_loop(0, b_per_w, mul_row, 0, unroll=4)

    pltpu.sync_copy(vb_v, outb_hbm.at[pl.ds(base, b_per_w)])
    pltpu.sync_copy(vw_v, outw_hbm.at[pl.ds(base, b_per_w)])


@jax.jit
def kernel(usuario, best_movie, worst_movie, W, V):
    B = usuario.shape[0]
    K = W.shape[1]
    b_per_w = B // NUM_WORKERS
    mesh = plsc.VectorSubcoreMesh(
        core_axis_name="c", subcore_axis_name="s",
        num_cores=NUM_CORES, num_subcores=NUM_SUBCORES)
    out_type = (
        jax.ShapeDtypeStruct((B, K), jnp.float32),
        jax.ShapeDtypeStruct((B, K), jnp.float32),
    )
    scratch = [
        pltpu.VMEM((b_per_w,), jnp.int32),
        pltpu.VMEM((b_per_w,), jnp.int32),
        pltpu.VMEM((b_per_w,), jnp.int32),
        pltpu.VMEM((b_per_w, K), jnp.float32),
        pltpu.VMEM((b_per_w, K), jnp.float32),
        pltpu.VMEM((b_per_w, K), jnp.float32),
        pltpu.SemaphoreType.DMA,
        pltpu.SemaphoreType.DMA,
        pltpu.SemaphoreType.DMA,
    ]
    fn = pl.kernel(
        functools.partial(_body, b_per_w, K),
        out_type=out_type,
        mesh=mesh,
        scratch_types=scratch,
        compiler_params=pltpu.CompilerParams(use_tc_tiling_on_sc=False),
    )
    return fn(usuario.reshape(B), best_movie.reshape(B),
              worst_movie.reshape(B), W, V)


# speedup vs baseline: 1.0011x; 1.0011x over previous
import functools

import jax
import jax.numpy as jnp
from jax import lax
from jax.experimental import pallas as pl
from jax.experimental.pallas import tpu as pltpu
from jax.experimental.pallas import tpu_sc as plsc

NUM_CORES = 2
NUM_SUBCORES = 16
NUM_WORKERS = NUM_CORES * NUM_SUBCORES
LANES = 16
PASSES = 2


def _body(b_per_w, K, u_hbm, b_hbm, w_hbm, W2_hbm, V2_hbm, out_hbm,
          uidx_v, bidx_v, widx_v, upar_v, bpar_v, wpar_v,
          wu_v, vb_v, vw_v, sem_u, sem_b, sem_w):
    wid = lax.axis_index("s") * NUM_CORES + lax.axis_index("c")
    base = wid * b_per_w
    rows = b_per_w // PASSES
    pltpu.sync_copy(u_hbm.at[pl.ds(base, b_per_w)], uidx_v)
    pltpu.sync_copy(b_hbm.at[pl.ds(base, b_per_w)], bidx_v)
    pltpu.sync_copy(w_hbm.at[pl.ds(base, b_per_w)], widx_v)

    for g in range(b_per_w // LANES):
        sl = pl.ds(g * LANES, LANES)
        for idx_v, par_v in ((uidx_v, upar_v), (bidx_v, bpar_v),
                             (widx_v, wpar_v)):
            v = idx_v[sl]
            par_v[g, pl.ds(0, LANES)] = (v & 1).astype(jnp.float32)
            idx_v[sl] = v >> 1

    for p in range(PASSES):
        cp_u = pltpu.async_copy(W2_hbm.at[uidx_v.at[pl.ds(p * rows, rows)]],
                                wu_v, sem_u)
        cp_b = pltpu.async_copy(V2_hbm.at[bidx_v.at[pl.ds(p * rows, rows)]],
                                vb_v, sem_b)
        cp_w = pltpu.async_copy(V2_hbm.at[widx_v.at[pl.ds(p * rows, rows)]],
                                vw_v, sem_w)
        cp_u.wait()
        cp_b.wait()
        cp_w.wait()

        def row(g, carry):
            lsl = pl.ds(0, LANES)
            pu16 = upar_v[p * (rows // LANES) + g, lsl]
            pb16 = bpar_v[p * (rows // LANES) + g, lsl]
            pw16 = wpar_v[p * (rows // LANES) + g, lsl]
            for l in range(LANES):
                r = g * LANES + l
                pu = pu16[l]
                pb = pb16[l]
                pw = pw16[l]
                for j in range(K // LANES):
                    lo = pl.ds(j * LANES, LANES)
                    hi = pl.ds(K + j * LANES, LANES)
                    wu_l = wu_v[r, lo]
                    wu = wu_l + pu * (wu_v[r, hi] - wu_l)
                    vb_l = vb_v[r, lo]
                    vb = vb_l + pb * (vb_v[r, hi] - vb_l)
                    vw_l = vw_v[r, lo]
                    vw = vw_l + pw * (vw_v[r, hi] - vw_l)
                    wu_v[r, lo] = wu * vb
                    wu_v[r, hi] = wu * vw
            return carry

        lax.fori_loop(0, rows // LANES, row, 0)

        pltpu.sync_copy(wu_v, out_hbm.at[pl.ds(base + p * rows, rows), :])


@jax.jit
def kernel(usuario, best_movie, worst_movie, W, V):
    B = usuario.shape[0]
    K = W.shape[1]
    b_per_w = B // NUM_WORKERS
    rows = b_per_w // PASSES
    W2 = W.reshape(W.shape[0] // 2, 2 * K)
    V2 = V.reshape(V.shape[0] // 2, 2 * K)
    mesh = plsc.VectorSubcoreMesh(
        core_axis_name="c", subcore_axis_name="s",
        num_cores=NUM_CORES, num_subcores=NUM_SUBCORES)
    out_type = jax.ShapeDtypeStruct((B, 2 * K), jnp.float32)
    scratch = [
        pltpu.VMEM((b_per_w,), jnp.int32),
        pltpu.VMEM((b_per_w,), jnp.int32),
        pltpu.VMEM((b_per_w,), jnp.int32),
        pltpu.VMEM((b_per_w // LANES, LANES), jnp.float32),
        pltpu.VMEM((b_per_w // LANES, LANES), jnp.float32),
        pltpu.VMEM((b_per_w // LANES, LANES), jnp.float32),
        pltpu.VMEM((rows, 2 * K), jnp.float32),
        pltpu.VMEM((rows, 2 * K), jnp.float32),
        pltpu.VMEM((rows, 2 * K), jnp.float32),
        pltpu.SemaphoreType.DMA,
        pltpu.SemaphoreType.DMA,
        pltpu.SemaphoreType.DMA,
    ]
    fn = pl.kernel(
        functools.partial(_body, b_per_w, K),
        out_type=out_type,
        mesh=mesh,
        scratch_types=scratch,
    )
    packed = fn(usuario.reshape(B), best_movie.reshape(B),
                worst_movie.reshape(B), W2, V2)
    return packed[:, :K], packed[:, K:]
